# Initial kernel scaffold; baseline (speedup 1.0000x reference)
import jax
import jax.numpy as jnp
from jax import lax
from jax.experimental import pallas as pl
from jax.experimental.pallas import tpu as pltpu
from jax.experimental.pallas import tpu_sc as plsc

B = 16384
N_FIELDS = 26
VOCAB = 100000
EMB_DIM = 32

NC = 2
NS = 16
NW = NC * NS
L = 16

R = B * N_FIELDS
RPW = R // NW
CHUNK = 1664
NCHUNK = RPW // CHUNK


def _emb_body(table_hbm, cat_hbm, out_hbm, catv, idxv, offv, rowsv, sem):
    wid = lax.axis_index("s") * NC + lax.axis_index("c")
    row0 = wid * RPW

    def off_body(j, carry):
        p = lax.iota(jnp.int32, L) + j * L
        offv[pl.ds(j * L, L)] = lax.rem(p, N_FIELDS) * VOCAB
        return carry

    lax.fori_loop(0, CHUNK // L, off_body, 0)

    def chunk_body(c, carry):
        base = row0 + c * CHUNK
        pltpu.sync_copy(cat_hbm.at[pl.ds(base, CHUNK)], catv)

        def idx_body(j, inner):
            s = pl.ds(j * L, L)
            idxv[s] = catv[s] + offv[s]
            return inner

        lax.fori_loop(0, CHUNK // L, idx_body, 0)
        pltpu.async_copy(table_hbm.at[idxv], rowsv, sem).wait()
        pltpu.sync_copy(rowsv, out_hbm.at[pl.ds(base, CHUNK)])
        return carry

    lax.fori_loop(0, NCHUNK, chunk_body, 0)


def kernel(continuous, categorical, emb_tables):
    table = emb_tables.reshape(N_FIELDS * VOCAB, EMB_DIM)
    cat_flat = categorical.reshape(R)
    mesh = plsc.VectorSubcoreMesh(core_axis_name="c", subcore_axis_name="s")
    out = pl.kernel(
        _emb_body,
        mesh=mesh,
        out_type=jax.ShapeDtypeStruct((R, EMB_DIM), jnp.float32),
        scratch_types=[
            pltpu.VMEM((CHUNK,), jnp.int32),
            pltpu.VMEM((CHUNK,), jnp.int32),
            pltpu.VMEM((CHUNK,), jnp.int32),
            pltpu.VMEM((CHUNK, EMB_DIM), jnp.float32),
            pltpu.SemaphoreType.DMA,
        ],
    )(table, cat_flat)
    return continuous, out.reshape(B, N_FIELDS * EMB_DIM)


# speedup vs baseline: 1.2068x; 1.2068x over previous
import jax
import jax.numpy as jnp
from jax import lax
from jax.experimental import pallas as pl
from jax.experimental.pallas import tpu as pltpu
from jax.experimental.pallas import tpu_sc as plsc

B = 16384
N_FIELDS = 26
VOCAB = 100000
EMB_DIM = 32

NC = 2
NS = 16
NW = NC * NS
L = 16

R = B * N_FIELDS
RPW = R // NW
CHUNK = 1664
NCHUNK = RPW // CHUNK


def _emb_body(table_hbm, cat_hbm, out_hbm, catv, idxv, offv, rowsv, sem):
    wid = lax.axis_index("s") * NC + lax.axis_index("c")
    row0 = wid * RPW

    def off_body(j, carry):
        p = lax.iota(jnp.int32, L) + j * L
        offv[pl.ds(j * L, L)] = lax.rem(p, N_FIELDS) * VOCAB
        return carry

    lax.fori_loop(0, CHUNK // L, off_body, 0)

    def chunk_body(c, carry):
        base = row0 + c * CHUNK
        pltpu.sync_copy(cat_hbm.at[pl.ds(base, CHUNK)], catv)

        def idx_body(j, inner):
            s = pl.ds(j * L, L)
            idxv[s] = catv[s] + offv[s]
            return inner

        lax.fori_loop(0, CHUNK // L, idx_body, 0)
        pltpu.async_copy(table_hbm.at[idxv], rowsv, sem).wait()
        pltpu.sync_copy(rowsv, out_hbm.at[pl.ds(base, CHUNK)])
        return carry

    lax.fori_loop(0, NCHUNK, chunk_body, 0)


def kernel(continuous, categorical, emb_tables):
    table = emb_tables.reshape(N_FIELDS * VOCAB, EMB_DIM)
    cat_flat = categorical.reshape(R)
    mesh = plsc.VectorSubcoreMesh(core_axis_name="c", subcore_axis_name="s")
    out = pl.kernel(
        _emb_body,
        mesh=mesh,
        compiler_params=pltpu.CompilerParams(use_tc_tiling_on_sc=False),
        out_type=jax.ShapeDtypeStruct((R, EMB_DIM), jnp.float32),
        scratch_types=[
            pltpu.VMEM((CHUNK,), jnp.int32),
            pltpu.VMEM((CHUNK,), jnp.int32),
            pltpu.VMEM((CHUNK,), jnp.int32),
            pltpu.VMEM((CHUNK, EMB_DIM), jnp.float32),
            pltpu.SemaphoreType.DMA,
        ],
    )(table, cat_flat)
    return continuous, out.reshape(B, N_FIELDS * EMB_DIM)


# speedup vs baseline: 4.0035x; 3.3174x over previous
import jax
import jax.numpy as jnp
from jax import lax
from jax.experimental import pallas as pl
from jax.experimental.pallas import tpu as pltpu
from jax.experimental.pallas import tpu_sc as plsc

B = 16384
N_FIELDS = 26
VOCAB = 100000
EMB_DIM = 32

NC = 2
NS = 16
NW = NC * NS
L = 16

F = N_FIELDS * EMB_DIM
FPW = F // NW
OCHUNK = 8192


def _emb_body(table_hbm, cat_hbm, out_hbm, rowv, idxv, outv, sem):
    wid = lax.axis_index("s") * NC + lax.axis_index("c")
    f0 = wid * FPW

    def do_row(f, _):
        pltpu.async_copy(table_hbm.at[f], rowv, sem).wait()

        def piece(p, __):
            def gather16(j, ___):
                s = pl.ds(p * OCHUNK + j * L, L)
                outv[pl.ds(j * L, L)] = plsc.load_gather(rowv, [idxv[s]])
                return ___

            lax.fori_loop(0, OCHUNK // L, gather16, 0, unroll=8)
            pltpu.sync_copy(outv, out_hbm.at[f, pl.ds(p * OCHUNK, OCHUNK)])
            return __

        lax.fori_loop(0, B // OCHUNK, piece, 0)
        return _

    iA = f0 // EMB_DIM
    iB = (f0 + FPW - 1) // EMB_DIM
    split = jnp.minimum((iA + 1) * EMB_DIM, f0 + FPW)

    pltpu.sync_copy(cat_hbm.at[iA], idxv)
    lax.fori_loop(f0, split, do_row, 0)
    pltpu.sync_copy(cat_hbm.at[iB], idxv)
    lax.fori_loop(split, f0 + FPW, do_row, 0)


def kernel(continuous, categorical, emb_tables):
    table_t = jnp.swapaxes(emb_tables, 1, 2).reshape(F, VOCAB)
    cat_t = categorical.T
    mesh = plsc.VectorSubcoreMesh(core_axis_name="c", subcore_axis_name="s")
    out_t = pl.kernel(
        _emb_body,
        mesh=mesh,
        compiler_params=pltpu.CompilerParams(
            use_tc_tiling_on_sc=True, needs_layout_passes=False
        ),
        out_type=jax.ShapeDtypeStruct((F, B), jnp.float32),
        scratch_types=[
            pltpu.VMEM((VOCAB,), jnp.float32),
            pltpu.VMEM((B,), jnp.int32),
            pltpu.VMEM((OCHUNK,), jnp.float32),
            pltpu.SemaphoreType.DMA,
        ],
    )(table_t, cat_t)
    return continuous, out_t.T


# speedup vs baseline: 8.3148x; 2.0769x over previous
import jax
import jax.numpy as jnp
from jax import lax
from jax.experimental import pallas as pl
from jax.experimental.pallas import tpu as pltpu
from jax.experimental.pallas import tpu_sc as plsc

B = 16384
N_FIELDS = 26
VOCAB = 100000
EMB_DIM = 32

NC = 2
NS = 16
NW = NC * NS
L = 16

F = N_FIELDS * EMB_DIM
FPW = F // NW
OP = 4096
NP = B // OP


def _emb_body(table_hbm, cat_hbm, out_hbm,
              rowv, idxv, outp0, outp1, semR, semO0, semO1):
    wid = lax.axis_index("s") * NC + lax.axis_index("c")
    f0 = wid * FPW
    outps = (outp0, outp1)
    semOs = (semO0, semO1)

    def make_row(first):
        def do_row(f, _):
            pltpu.make_async_copy(table_hbm.at[f], rowv, semR).wait()
            for p in range(NP):
                ob, so = outps[p % 2], semOs[p % 2]
                if not (first and p < 2):
                    pltpu.make_async_copy(
                        ob, out_hbm.at[f, pl.ds(p * OP, OP)], so).wait()

                @plsc.parallel_loop(0, OP // L, unroll=8)
                def gather16(j):
                    s = pl.ds(p * OP + j * L, L)
                    ob[pl.ds(j * L, L)] = plsc.load_gather(rowv, [idxv[s]])

                pltpu.async_copy(ob, out_hbm.at[f, pl.ds(p * OP, OP)], so)
            fn = jnp.minimum(f + 1, F - 1)
            pltpu.async_copy(table_hbm.at[fn], rowv, semR)
            return _

        return do_row

    iA = f0 // EMB_DIM
    iB = (f0 + FPW - 1) // EMB_DIM
    split = jnp.minimum((iA + 1) * EMB_DIM, f0 + FPW)

    pltpu.async_copy(table_hbm.at[f0], rowv, semR)
    pltpu.sync_copy(cat_hbm.at[iA], idxv)
    make_row(True)(f0, 0)
    lax.fori_loop(f0 + 1, split, make_row(False), 0)
    pltpu.sync_copy(cat_hbm.at[iB], idxv)
    lax.fori_loop(split, f0 + FPW, make_row(False), 0)

    pltpu.make_async_copy(table_hbm.at[f0], rowv, semR).wait()
    pltpu.make_async_copy(outp0, out_hbm.at[f0, pl.ds(0, OP)], semO0).wait()
    pltpu.make_async_copy(outp1, out_hbm.at[f0, pl.ds(OP, OP)], semO1).wait()


def kernel(continuous, categorical, emb_tables):
    table_t = jnp.swapaxes(emb_tables, 1, 2).reshape(F, VOCAB)
    cat_t = categorical.T
    mesh = plsc.VectorSubcoreMesh(core_axis_name="c", subcore_axis_name="s")
    out_t = pl.kernel(
        _emb_body,
        mesh=mesh,
        compiler_params=pltpu.CompilerParams(
            use_tc_tiling_on_sc=True, needs_layout_passes=False
        ),
        out_type=jax.ShapeDtypeStruct((F, B), jnp.float32),
        scratch_types=[
            pltpu.VMEM((VOCAB,), jnp.float32),
            pltpu.VMEM((B,), jnp.int32),
            pltpu.VMEM((OP,), jnp.float32),
            pltpu.VMEM((OP,), jnp.float32),
            pltpu.SemaphoreType.DMA,
            pltpu.SemaphoreType.DMA,
            pltpu.SemaphoreType.DMA,
        ],
    )(table_t, cat_t)
    return continuous, out_t.T
